# baseline (speedup 1.0000x reference)
import functools

import jax
import jax.numpy as jnp
from jax import lax
from jax.experimental import pallas as pl
from jax.experimental.pallas import tpu as pltpu
from jax.experimental.pallas import tpu_sc as plsc

_SEQ = 8192
_W = 32
_DIM = _W * _W

_NDMA = 4


def _sc_body(table_hbm, out_hbm, *sems):
    info = plsc.get_sparse_core_info()
    nw = info.num_cores * info.num_subcores
    rows = _SEQ // nw
    chunk = rows // _NDMA
    wid = lax.axis_index("s") * info.num_cores + lax.axis_index("c")
    base = wid * rows

    copies = [
        pltpu.make_async_copy(
            table_hbm.at[pl.ds(base + d * chunk, chunk)],
            out_hbm.at[pl.ds(base + d * chunk, chunk)],
            sems[d],
        )
        for d in range(_NDMA)
    ]
    for c in copies:
        c.start()
    for c in copies:
        c.wait()


def kernel(x, position_embeddings):
    del x
    mesh = plsc.VectorSubcoreMesh(core_axis_name="c", subcore_axis_name="s")
    sc_copy = functools.partial(
        pl.kernel,
        mesh=mesh,
        out_type=jax.ShapeDtypeStruct((_SEQ, _DIM), jnp.float32),
        scratch_types=[pltpu.SemaphoreType.DMA for _ in range(_NDMA)],
    )(_sc_body)
    out = sc_copy(position_embeddings)
    return out.reshape(1, _SEQ, _W, _W)


# speedup vs baseline: 14.1002x; 14.1002x over previous
import functools

import jax
import jax.numpy as jnp
from jax import lax
from jax.experimental import pallas as pl
from jax.experimental.pallas import tpu as pltpu
from jax.experimental.pallas import tpu_sc as plsc

_SEQ = 8192
_W = 32
_DIM = _W * _W

_NBUF = 3
_CHUNK = 512


def _copy_body(table_hbm, out_hbm, *scratch):
    bufs = scratch[:_NBUF]
    sin = scratch[_NBUF:2 * _NBUF]
    sout = scratch[2 * _NBUF:]
    info = plsc.get_sparse_core_info()
    rows = _SEQ // info.num_cores
    nchunks = rows // _CHUNK
    base = lax.axis_index("c") * rows

    def in_copy(b, c):
        return pltpu.make_async_copy(
            table_hbm.at[pl.ds(base + c * _CHUNK, _CHUNK)], bufs[b], sin[b])

    def out_copy(b, c):
        return pltpu.make_async_copy(
            bufs[b], out_hbm.at[pl.ds(base + c * _CHUNK, _CHUNK)], sout[b])

    for b in range(_NBUF):
        in_copy(b, b).start()
    for c in range(nchunks):
        b = c % _NBUF
        in_copy(b, c).wait()
        out_copy(b, c).start()
        nxt = c + _NBUF
        if nxt < nchunks:
            out_copy(b, c).wait()
            in_copy(b, nxt).start()
    for c in range(max(0, nchunks - _NBUF), nchunks):
        out_copy(c % _NBUF, c).wait()


def kernel(x, position_embeddings):
    del x
    mesh = plsc.ScalarSubcoreMesh(axis_name="c", num_cores=2)
    copy = functools.partial(
        pl.kernel,
        mesh=mesh,
        out_type=jax.ShapeDtypeStruct((_SEQ, _DIM), jnp.float32),
        scratch_types=(
            [pltpu.VMEM_SHARED((_CHUNK, _DIM), jnp.float32)
             for _ in range(_NBUF)]
            + [pltpu.SemaphoreType.DMA for _ in range(2 * _NBUF)]
        ),
    )(_copy_body)
    out = copy(position_embeddings)
    return out.reshape(1, _SEQ, _W, _W)


# speedup vs baseline: 14.6329x; 1.0378x over previous
import functools

import jax
import jax.numpy as jnp
from jax import lax
from jax.experimental import pallas as pl
from jax.experimental.pallas import tpu as pltpu
from jax.experimental.pallas import tpu_sc as plsc

_SEQ = 8192
_W = 32
_DIM = _W * _W

_NBUF = 3
_CHUNK = 32


def _sc_body(table_hbm, out_hbm, *scratch):
    bufs = scratch[:_NBUF]
    sin = scratch[_NBUF:2 * _NBUF]
    sout = scratch[2 * _NBUF:]
    info = plsc.get_sparse_core_info()
    nw = info.num_cores * info.num_subcores
    rows = _SEQ // nw
    nchunks = rows // _CHUNK
    wid = lax.axis_index("s") * info.num_cores + lax.axis_index("c")
    base = wid * rows

    def in_copy(b, c):
        return pltpu.make_async_copy(
            table_hbm.at[pl.ds(base + c * _CHUNK, _CHUNK)], bufs[b], sin[b])

    def out_copy(b, c):
        return pltpu.make_async_copy(
            bufs[b], out_hbm.at[pl.ds(base + c * _CHUNK, _CHUNK)], sout[b])

    for b in range(min(_NBUF, nchunks)):
        in_copy(b, b).start()
    for c in range(nchunks):
        b = c % _NBUF
        in_copy(b, c).wait()
        out_copy(b, c).start()
        nxt = c + _NBUF
        if nxt < nchunks:
            out_copy(b, c).wait()
            in_copy(b, nxt).start()
    for c in range(max(0, nchunks - _NBUF), nchunks):
        out_copy(c % _NBUF, c).wait()


def kernel(x, position_embeddings):
    del x
    mesh = plsc.VectorSubcoreMesh(core_axis_name="c", subcore_axis_name="s")
    sc_copy = functools.partial(
        pl.kernel,
        mesh=mesh,
        out_type=jax.ShapeDtypeStruct((_SEQ, _DIM), jnp.float32),
        scratch_types=(
            [pltpu.VMEM((_CHUNK, _DIM), jnp.float32) for _ in range(_NBUF)]
            + [pltpu.SemaphoreType.DMA for _ in range(2 * _NBUF)]
        ),
    )(_sc_body)
    out = sc_copy(position_embeddings)
    return out.reshape(1, _SEQ, _W, _W)
